# baseline (speedup 1.0000x reference)
import jax
import jax.numpy as jnp
from jax.experimental import pallas as pl

M_SUB = 16
K_CODES = 512
D_SUB = 64


def _matmul_kernel(x_ref, w_ref, out_ref):
    xb = x_ref[...].astype(jnp.bfloat16)
    wb = w_ref[...].astype(jnp.bfloat16)
    out_ref[...] = jax.lax.dot_general(
        xb, wb, (((1,), (1,)), ((), ())),
        preferred_element_type=jnp.float32)


def _pq_kernel(x_ref, cb_ref, cbt_ref, xq_ref):
    B = x_ref.shape[0]
    k_iota = jax.lax.broadcasted_iota(jnp.int32, (B, K_CODES), 1)
    for m in range(M_SUB):
        xs = x_ref[:, m * D_SUB:(m + 1) * D_SUB]
        cbt = cbt_ref[m]
        c2 = jnp.sum(cbt * cbt, axis=0)
        xc = jnp.dot(xs, cbt,
                     preferred_element_type=jnp.float32)
        d = c2[None, :] - 2.0 * xc
        min_d = jnp.min(d, axis=1, keepdims=True)
        idx = jnp.min(jnp.where(d == min_d, k_iota, K_CODES),
                      axis=1, keepdims=True)
        onehot = (k_iota == idx).astype(jnp.bfloat16)
        xq_ref[:, m * D_SUB:(m + 1) * D_SUB] = jnp.dot(
            onehot, cb_ref[m].astype(jnp.bfloat16),
            preferred_element_type=jnp.float32)


def kernel(x, weight, codebooks):
    N, D = x.shape
    OUT = weight.shape[0]

    BM = 512
    out = pl.pallas_call(
        _matmul_kernel,
        grid=(N // BM,),
        in_specs=[
            pl.BlockSpec((BM, D), lambda i: (i, 0)),
            pl.BlockSpec((OUT, D), lambda i: (0, 0)),
        ],
        out_specs=pl.BlockSpec((BM, OUT), lambda i: (i, 0)),
        out_shape=jax.ShapeDtypeStruct((N, OUT), jnp.float32),
    )(x, weight)

    BQ = 256
    cbt = jnp.swapaxes(codebooks, 1, 2)
    xq = pl.pallas_call(
        _pq_kernel,
        grid=(N // BQ,),
        in_specs=[
            pl.BlockSpec((BQ, D), lambda i: (i, 0)),
            pl.BlockSpec((M_SUB, K_CODES, D_SUB), lambda i: (0, 0, 0)),
            pl.BlockSpec((M_SUB, D_SUB, K_CODES), lambda i: (0, 0, 0)),
        ],
        out_specs=pl.BlockSpec((BQ, D), lambda i: (i, 0)),
        out_shape=jax.ShapeDtypeStruct((N, D), jnp.float32),
    )(x, codebooks, cbt)

    return (out, xq)


# speedup vs baseline: 1.3996x; 1.3996x over previous
import jax
import jax.numpy as jnp
from jax.experimental import pallas as pl

M_SUB = 16
K_CODES = 512
D_SUB = 64


def _matmul_kernel(x_ref, w_ref, out_ref):
    xb = x_ref[...].astype(jnp.bfloat16)
    wb = w_ref[...].astype(jnp.bfloat16)
    out_ref[...] = jax.lax.dot_general(
        xb, wb, (((1,), (1,)), ((), ())),
        preferred_element_type=jnp.float32)


def _pq_kernel(x_ref, cb_ref, cbt_ref, xq_ref):
    B = x_ref.shape[0]
    k_iota = jax.lax.broadcasted_iota(jnp.int32, (B, K_CODES), 1)
    for m in range(M_SUB):
        xs = x_ref[:, m * D_SUB:(m + 1) * D_SUB]
        cbt = cbt_ref[m]
        c2 = jnp.sum(cbt * cbt, axis=0)
        xc = jnp.dot(xs, cbt,
                     preferred_element_type=jnp.float32)
        d = c2[None, :] - 2.0 * xc
        min_d = jnp.min(d, axis=1, keepdims=True)
        idx = jnp.min(jnp.where(d == min_d, k_iota, K_CODES),
                      axis=1, keepdims=True)
        onehot = (k_iota == idx).astype(jnp.float32)
        xq_ref[:, m * D_SUB:(m + 1) * D_SUB] = jnp.dot(
            onehot, cb_ref[m], preferred_element_type=jnp.float32)


def kernel(x, weight, codebooks):
    N, D = x.shape
    OUT = weight.shape[0]

    BM = 512
    out = pl.pallas_call(
        _matmul_kernel,
        grid=(N // BM,),
        in_specs=[
            pl.BlockSpec((BM, D), lambda i: (i, 0)),
            pl.BlockSpec((OUT, D), lambda i: (0, 0)),
        ],
        out_specs=pl.BlockSpec((BM, OUT), lambda i: (i, 0)),
        out_shape=jax.ShapeDtypeStruct((N, OUT), jnp.float32),
    )(x, weight)

    BQ = 256
    cbt = jnp.swapaxes(codebooks, 1, 2)
    xq = pl.pallas_call(
        _pq_kernel,
        grid=(N // BQ,),
        in_specs=[
            pl.BlockSpec((BQ, D), lambda i: (i, 0)),
            pl.BlockSpec((M_SUB, K_CODES, D_SUB), lambda i: (0, 0, 0)),
            pl.BlockSpec((M_SUB, D_SUB, K_CODES), lambda i: (0, 0, 0)),
        ],
        out_specs=pl.BlockSpec((BQ, D), lambda i: (i, 0)),
        out_shape=jax.ShapeDtypeStruct((N, D), jnp.float32),
    )(x, codebooks, cbt)

    return (out, xq)
